# Initial kernel scaffold; baseline (speedup 1.0000x reference)
import jax
import jax.numpy as jnp
from jax.experimental import pallas as pl


def kernel(x, edge_index, edge_type, basis, comp, root, conv_bias, Wg, bg, Ws, bs):
    raise NotImplementedError("write your pallas kernel here")


# speedup vs baseline: 74.8270x; 74.8270x over previous
import functools

import jax
import jax.numpy as jnp
from jax import lax
from jax.experimental import pallas as pl
from jax.experimental.pallas import tpu as pltpu
from jax.experimental.pallas import tpu_sc as plsc

N = 10000
D = 128
E = 320000
R = 5
G_S = 40000
S_ = 20000

NC = 2
NS = 16
NW = NC * NS
EPW = E // NW
LANES = 16
NVREG = EPW // LANES
UNROLL = 5
NGROUP = NVREG // UNROLL
ACC_W = D + LANES


def _sc_edge_filter(src_hbm, dst_hbm, typ_hbm, x_hbm, acc_out,
                    dstv, srcv, typv, acc, idxv, rows, sem):
    wid = lax.axis_index("s") * NC + lax.axis_index("c")
    base = wid * EPW
    pltpu.sync_copy(dst_hbm.at[pl.ds(base, EPW)], dstv)
    pltpu.sync_copy(src_hbm.at[pl.ds(base, EPW)], srcv)
    pltpu.sync_copy(typ_hbm.at[pl.ds(base, EPW)], typv)

    zf = jnp.zeros((LANES,), jnp.float32)
    for r in range(R):
        for c in range(ACC_W // LANES):
            acc[r, pl.ds(c * LANES, LANES)] = zf
    ones = jnp.ones((LANES,), jnp.float32)
    lanes_iota = lax.broadcasted_iota(jnp.int32, (LANES,), 0)

    def handle_vreg(off):
        d = dstv[pl.ds(off, LANES)]
        m = d == 0
        nh = jnp.sum(jnp.where(m, 1, 0))

        @pl.when(nh > 0)
        def _():
            s = srcv[pl.ds(off, LANES)]
            t = typv[pl.ds(off, LANES)]
            idxv[...] = jnp.where(m, s, 0)
            pltpu.async_copy(x_hbm.at[idxv], rows, sem).wait()

            def lane_body(l, carry):
                sel = lanes_iota == l
                hit = jnp.sum(jnp.where(sel & m, 1, 0))

                @pl.when(hit > 0)
                def _():
                    t_l = jnp.sum(jnp.where(sel, t, 0))
                    for c in range(D // LANES):
                        acc[t_l, pl.ds(c * LANES, LANES)] += (
                            rows[l, pl.ds(c * LANES, LANES)])
                    acc[t_l, pl.ds(D, LANES)] += ones

                return carry

            lax.fori_loop(0, LANES, lane_body, 0)

    def group_body(g, carry):
        b = pl.multiple_of(g * (UNROLL * LANES), LANES)
        mn = dstv[pl.ds(b, LANES)]
        for j in range(1, UNROLL):
            mn = jnp.minimum(mn, dstv[pl.ds(b + j * LANES, LANES)])
        any0 = jnp.sum(jnp.where(mn == 0, 1, 0))

        @pl.when(any0 > 0)
        def _():
            def vbody(j, c2):
                off = pl.multiple_of(b + j * LANES, LANES)
                handle_vreg(off)
                return c2

            lax.fori_loop(0, UNROLL, vbody, 0)

        return carry

    lax.fori_loop(0, NGROUP, group_body, 0)
    pltpu.sync_copy(acc, acc_out.at[wid])


_sc_filter_call = functools.partial(
    pl.kernel,
    out_type=jax.ShapeDtypeStruct((NW, R, ACC_W), jnp.float32),
    mesh=plsc.VectorSubcoreMesh(core_axis_name="c", subcore_axis_name="s"),
    compiler_params=pltpu.CompilerParams(needs_layout_passes=False),
    scratch_types=[
        pltpu.VMEM((EPW,), jnp.int32),
        pltpu.VMEM((EPW,), jnp.int32),
        pltpu.VMEM((EPW,), jnp.int32),
        pltpu.VMEM((R, ACC_W), jnp.float32),
        pltpu.VMEM((LANES,), jnp.int32),
        pltpu.VMEM((LANES, D), jnp.float32),
        pltpu.SemaphoreType.DMA,
    ],
)(_sc_edge_filter)


def _tc_dense(x0_ref, root_ref, bias_ref, basis_ref, comp_ref, acc_ref,
              wg_ref, bg_ref, ws_ref, bs_ref, outg_ref, outs_ref):
    accs = acc_ref[...]
    s = jnp.sum(accs[:, :, :D], axis=0)
    cnt = jnp.sum(accs[:, :, D:D + 1], axis=0)
    u = s / jnp.maximum(cnt, 1.0)
    v = jnp.einsum('rb,rd->bd', comp_ref[...], u)
    basis2 = basis_ref[...].reshape(R * D, D)
    msg = v.reshape(1, R * D) @ basis2
    out0 = x0_ref[...] @ root_ref[...] + bias_ref[...] + msg
    x0 = jnp.maximum(out0, 0.0)

    zg = x0 @ wg_ref[...] + bg_ref[...]
    mg = jnp.max(zg)
    lg = jnp.log(jnp.sum(jnp.exp(zg - mg)))
    outg_ref[...] = zg - mg - lg

    zs = x0 @ ws_ref[...] + bs_ref[...]
    ms = jnp.max(zs)
    ls = jnp.log(jnp.sum(jnp.exp(zs - ms)))
    outs_ref[...] = zs - ms - ls


_tc_dense_call = pl.pallas_call(
    _tc_dense,
    out_shape=[
        jax.ShapeDtypeStruct((1, G_S), jnp.float32),
        jax.ShapeDtypeStruct((1, S_), jnp.float32),
    ],
)


@jax.jit
def kernel(x, edge_index, edge_type, basis, comp, root, conv_bias,
           Wg, bg, Ws, bs):
    src = edge_index[0]
    dst = edge_index[1]
    acc = _sc_filter_call(src, dst, edge_type, x)
    outg, outs = _tc_dense_call(
        x[0:1], root, conv_bias.reshape(1, D), basis, comp, acc,
        Wg, bg.reshape(1, G_S), Ws, bs.reshape(1, S_))
    return outg.reshape(G_S), outs.reshape(S_)


# speedup vs baseline: 118.9134x; 1.5892x over previous
import functools

import jax
import jax.numpy as jnp
from jax import lax
from jax.experimental import pallas as pl
from jax.experimental.pallas import tpu as pltpu
from jax.experimental.pallas import tpu_sc as plsc

N = 10000
D = 128
E = 320000
R = 5
G_S = 40000
S_ = 20000

NC = 2
NS = 16
NW = NC * NS
EPW = E // NW
LANES = 16
NVREG = EPW // LANES
UNROLL = 25
NGROUP = NVREG // UNROLL
ACC_W = D + LANES


def _sc_edge_filter(ei_hbm, typ_hbm, x_hbm, acc_out,
                    dstv, acc, idxv, src16, typ16, rows, sem):
    wid = lax.axis_index("s") * NC + lax.axis_index("c")
    base = wid * EPW
    pltpu.sync_copy(ei_hbm.at[1, pl.ds(base, EPW)], dstv)

    zf = jnp.zeros((LANES,), jnp.float32)
    for r in range(R):
        for c in range(ACC_W // LANES):
            acc[r, pl.ds(c * LANES, LANES)] = zf
    ones = jnp.ones((LANES,), jnp.float32)
    lanes_iota = lax.broadcasted_iota(jnp.int32, (LANES,), 0)

    def handle_vreg(off):
        d = dstv[pl.ds(off, LANES)]
        m = d == 0
        nh = jnp.sum(jnp.where(m, 1, 0))

        @pl.when(nh > 0)
        def _():
            pltpu.sync_copy(ei_hbm.at[0, pl.ds(base + off, LANES)], src16)
            pltpu.sync_copy(typ_hbm.at[pl.ds(base + off, LANES)], typ16)
            t = typ16[...]
            idxv[...] = jnp.where(m, src16[...], 0)
            pltpu.async_copy(x_hbm.at[idxv], rows, sem).wait()

            def lane_body(l, carry):
                sel = lanes_iota == l
                hit = jnp.sum(jnp.where(sel & m, 1, 0))

                @pl.when(hit > 0)
                def _():
                    t_l = jnp.sum(jnp.where(sel, t, 0))
                    for c in range(D // LANES):
                        acc[t_l, pl.ds(c * LANES, LANES)] += (
                            rows[l, pl.ds(c * LANES, LANES)])
                    acc[t_l, pl.ds(D, LANES)] += ones

                return carry

            lax.fori_loop(0, LANES, lane_body, 0)

    def group_body(g, carry):
        b = pl.multiple_of(g * (UNROLL * LANES), LANES)
        mn = dstv[pl.ds(b, LANES)]
        for j in range(1, UNROLL):
            mn = jnp.minimum(mn, dstv[pl.ds(b + j * LANES, LANES)])
        any0 = jnp.sum(jnp.where(mn == 0, 1, 0))

        @pl.when(any0 > 0)
        def _():
            def vbody(j, c2):
                off = pl.multiple_of(b + j * LANES, LANES)
                handle_vreg(off)
                return c2

            lax.fori_loop(0, UNROLL, vbody, 0)

        return carry

    lax.fori_loop(0, NGROUP, group_body, 0)
    pltpu.sync_copy(acc, acc_out.at[wid])


_sc_filter_call = functools.partial(
    pl.kernel,
    out_type=jax.ShapeDtypeStruct((NW, R, ACC_W), jnp.float32),
    mesh=plsc.VectorSubcoreMesh(core_axis_name="c", subcore_axis_name="s"),
    compiler_params=pltpu.CompilerParams(
        needs_layout_passes=False, use_tc_tiling_on_sc=False),
    scratch_types=[
        pltpu.VMEM((EPW,), jnp.int32),
        pltpu.VMEM((R, ACC_W), jnp.float32),
        pltpu.VMEM((LANES,), jnp.int32),
        pltpu.VMEM((LANES,), jnp.int32),
        pltpu.VMEM((LANES,), jnp.int32),
        pltpu.VMEM((LANES, D), jnp.float32),
        pltpu.SemaphoreType.DMA,
    ],
)(_sc_edge_filter)


def _tc_dense(x0_ref, root_ref, bias_ref, basis_ref, comp_ref, acc_ref,
              wgt_ref, bg_ref, wst_ref, bs_ref, outg_ref, outs_ref):
    accs = acc_ref[...]
    s = jnp.sum(accs[:, :, :D], axis=0)
    cnt = jnp.sum(accs[:, :, D:D + 1], axis=0)
    u = s / jnp.maximum(cnt, 1.0)
    v = jnp.einsum('rb,rd->bd', comp_ref[...], u)
    basis2 = basis_ref[...].reshape(R * D, D)
    msg = v.reshape(1, R * D) @ basis2
    out0 = x0_ref[...] @ root_ref[...] + bias_ref[...] + msg
    x0 = jnp.maximum(out0, 0.0)

    dn = (((1,), (1,)), ((), ()))
    zg = lax.dot_general(x0, wgt_ref[...], dn) + bg_ref[...]
    mg = jnp.max(zg)
    lg = jnp.log(jnp.sum(jnp.exp(zg - mg)))
    outg_ref[...] = (zg - mg - lg).reshape(G_S)

    zs = lax.dot_general(x0, wst_ref[...], dn) + bs_ref[...]
    ms = jnp.max(zs)
    ls = jnp.log(jnp.sum(jnp.exp(zs - ms)))
    outs_ref[...] = (zs - ms - ls).reshape(S_)


_tc_dense_call = pl.pallas_call(
    _tc_dense,
    out_shape=[
        jax.ShapeDtypeStruct((G_S,), jnp.float32),
        jax.ShapeDtypeStruct((S_,), jnp.float32),
    ],
)


@jax.jit
def kernel(x, edge_index, edge_type, basis, comp, root, conv_bias,
           Wg, bg, Ws, bs):
    acc = _sc_filter_call(edge_index, edge_type, x)
    outg, outs = _tc_dense_call(
        x[0:1], root, conv_bias.reshape(1, D), basis, comp, acc,
        Wg.T, bg.reshape(1, G_S), Ws.T, bs.reshape(1, S_))
    return outg, outs
